# initial kernel scaffold (unmeasured)
import jax
import jax.numpy as jnp
from jax import lax
from jax.experimental import pallas as pl
from jax.experimental.pallas import tpu as pltpu

F32 = jnp.float32
BF16 = jnp.bfloat16

B, S, D = 4, 256, 4096
H, DH, DR = 32, 128, 64
DC_SH = 128
SCALE = (DH + DR) ** -0.5


def _ring_coords(rr):
    cx = jnp.where(rr >= 2, 1, 0)
    cy = jnp.where((rr == 1) | (rr == 2), 1, 0)
    return cx, cy


def _kv_body(xb_ref, xp_ref, wdkv_ref, wuk_ref, wuv_ref, k_ref, v_ref,
             send_buf, recv_buf, send_sem, recv_sem):
    my_x = lax.axis_index("x")
    my_y = lax.axis_index("y")
    wdkv = wdkv_ref[...].astype(BF16)
    wuk = wuk_ref[...].astype(BF16)
    wuv = wuv_ref[...].astype(BF16)
    cp = jnp.dot(xp_ref[...], wdkv, preferred_element_type=F32).astype(BF16)
    send_buf[0, :, :] = jnp.dot(cp, wuk, preferred_element_type=F32).astype(BF16)
    send_buf[1, :, :] = jnp.dot(cp, wuv, preferred_element_type=F32).astype(BF16)
    rdma = pltpu.make_async_remote_copy(
        src_ref=send_buf,
        dst_ref=recv_buf,
        send_sem=send_sem,
        recv_sem=recv_sem,
        device_id=(1 - my_x, my_y),
        device_id_type=pl.DeviceIdType.MESH,
    )
    rdma.start()
    cm = jnp.dot(xb_ref[...], wdkv, preferred_element_type=F32).astype(BF16)
    kp = jnp.dot(cm, wuk, preferred_element_type=F32)
    vp = jnp.dot(cm, wuv, preferred_element_type=F32)
    rdma.wait()
    k_ref[...] = (kp + recv_buf[0, :, :].astype(F32)).astype(BF16)
    v_ref[...] = (vp + recv_buf[1, :, :].astype(F32)).astype(BF16)


def _kv(xb_bf, xp_bf, Wdkv, Wuk, Wuv):
    return pl.pallas_call(
        _kv_body,
        out_shape=[
            jax.ShapeDtypeStruct((S, D), BF16),
            jax.ShapeDtypeStruct((S, D), BF16),
        ],
        in_specs=[pl.BlockSpec(memory_space=pltpu.VMEM)] * 5,
        out_specs=[pl.BlockSpec(memory_space=pltpu.VMEM)] * 2,
        scratch_shapes=[
            pltpu.VMEM((2, S, D), BF16),
            pltpu.VMEM((2, S, D), BF16),
            pltpu.SemaphoreType.DMA,
            pltpu.SemaphoreType.DMA,
        ],
        compiler_params=pltpu.CompilerParams(collective_id=1),
    )(xb_bf, xp_bf, Wdkv, Wuk, Wuv)


def _gemm_body(x_ref, w_ref, o_ref):
    o_ref[...] = jnp.dot(
        x_ref[...], w_ref[...].astype(BF16), preferred_element_type=F32
    ).astype(BF16)


def _gemm(xb_bf, W, blk=512):
    n = W.shape[1]
    return pl.pallas_call(
        _gemm_body,
        grid=(n // blk,),
        in_specs=[
            pl.BlockSpec((S, D), lambda j: (0, 0)),
            pl.BlockSpec((D, blk), lambda j: (0, j)),
        ],
        out_specs=pl.BlockSpec((S, blk), lambda j: (0, j)),
        out_shape=jax.ShapeDtypeStruct((S, n), BF16),
    )(xb_bf, W)


def _qr_kr_body(x_ref, wqr_ref, wkr_ref, qr_ref, kr_ref):
    x = x_ref[...]
    qr_ref[...] = jnp.dot(
        x, wqr_ref[...].astype(BF16), preferred_element_type=F32
    ).astype(BF16)
    kr_ref[...] = jnp.dot(
        x, wkr_ref[...].astype(BF16), preferred_element_type=F32
    ).astype(BF16)


def _qr_kr(xb_bf, Wqr, Wkr, blk=512):
    return pl.pallas_call(
        _qr_kr_body,
        grid=(Wqr.shape[1] // blk,),
        in_specs=[
            pl.BlockSpec((S, D), lambda j: (0, 0)),
            pl.BlockSpec((D, blk), lambda j: (0, j)),
            pl.BlockSpec((D, DR), lambda j: (0, 0)),
        ],
        out_specs=[
            pl.BlockSpec((S, blk), lambda j: (0, j)),
            pl.BlockSpec((S, DR), lambda j: (0, 0)),
        ],
        out_shape=[
            jax.ShapeDtypeStruct((S, Wqr.shape[1]), BF16),
            jax.ShapeDtypeStruct((S, DR), BF16),
        ],
    )(xb_bf, Wqr, Wkr)


def _attn_body(q_ref, k_ref, v_ref, qr_ref, kr_ref, o_ref):
    qh = q_ref[:, 0, :]
    kh = k_ref[:, 0, :]
    vh = v_ref[:, 0, :]
    qrh = qr_ref[:, 0, :]
    krh = kr_ref[...]
    s = lax.dot_general(qh, kh, (((1,), (1,)), ((), ())),
                        preferred_element_type=F32)
    s = s + lax.dot_general(qrh, krh, (((1,), (1,)), ((), ())),
                            preferred_element_type=F32)
    s = s * SCALE
    m = jnp.max(s, axis=1, keepdims=True)
    e = jnp.exp(s - m)
    p = (e / jnp.sum(e, axis=1, keepdims=True)).astype(BF16)
    o_ref[:, 0, :] = jnp.dot(p, vh, preferred_element_type=F32).astype(BF16)


def _attn(q, k, v, qr, kr):
    return pl.pallas_call(
        _attn_body,
        grid=(H,),
        in_specs=[
            pl.BlockSpec((S, 1, DH), lambda h: (0, h, 0)),
            pl.BlockSpec((S, 1, DH), lambda h: (0, h, 0)),
            pl.BlockSpec((S, 1, DH), lambda h: (0, h, 0)),
            pl.BlockSpec((S, 1, DR), lambda h: (0, h, 0)),
            pl.BlockSpec((S, DR), lambda h: (0, 0)),
        ],
        out_specs=pl.BlockSpec((S, 1, DH), lambda h: (0, h, 0)),
        out_shape=jax.ShapeDtypeStruct((S, H, DH), BF16),
    )(q, k, v, qr, kr)


def _ag_body(in_ref, out_ref, comm, send_sems, recv_sems):
    my_x = lax.axis_index("x")
    my_y = lax.axis_index("y")
    r = jnp.where(my_x == 0, my_y, 3 - my_y)
    right = _ring_coords((r + 1) % 4)
    left = _ring_coords((r + 3) % 4)

    barrier = pltpu.get_barrier_semaphore()
    for nbr in (left, right):
        pl.semaphore_signal(barrier, inc=1, device_id=nbr,
                            device_id_type=pl.DeviceIdType.MESH)
    pl.semaphore_wait(barrier, 2)

    comm[0, :, :] = in_ref[...]
    out_ref[pl.ds(r, 1), :, :] = in_ref[...].astype(F32).reshape(1, S, D)

    for h in range(3):
        rdma = pltpu.make_async_remote_copy(
            src_ref=comm.at[h],
            dst_ref=comm.at[h + 1],
            send_sem=send_sems.at[h],
            recv_sem=recv_sems.at[h],
            device_id=right,
            device_id_type=pl.DeviceIdType.MESH,
        )
        rdma.start()
        rdma.wait()
        origin = (r - h - 1) % 4
        out_ref[pl.ds(origin, 1), :, :] = (
            comm[h + 1, :, :].astype(F32).reshape(1, S, D)
        )


def _all_gather(out_me):
    return pl.pallas_call(
        _ag_body,
        out_shape=jax.ShapeDtypeStruct((B, S, D), F32),
        in_specs=[pl.BlockSpec(memory_space=pltpu.VMEM)],
        out_specs=pl.BlockSpec(memory_space=pltpu.VMEM),
        scratch_shapes=[
            pltpu.VMEM((4, S, D), BF16),
            pltpu.SemaphoreType.DMA((3,)),
            pltpu.SemaphoreType.DMA((3,)),
        ],
        compiler_params=pltpu.CompilerParams(collective_id=0),
    )(out_me)


def kernel(x, Wdkv, Wuk, Wuv, Wq, Wqr, Wkr, Wo):
    my_x = lax.axis_index("x")
    my_y = lax.axis_index("y")
    r = jnp.where(my_x == 0, my_y, 3 - my_y)
    pr = jnp.where(my_x == 1, my_y, 3 - my_y)

    xb_bf = lax.dynamic_index_in_dim(x, r, 0, keepdims=False).astype(BF16)
    xp_bf = lax.dynamic_index_in_dim(x, pr, 0, keepdims=False).astype(BF16)

    k_flat, v_flat = _kv(xb_bf, xp_bf, Wdkv, Wuk, Wuv)
    q_flat = _gemm(xb_bf, Wq)
    qr_flat, kr = _qr_kr(xb_bf, Wqr, Wkr)

    o = _attn(
        q_flat.reshape(S, H, DH),
        k_flat.reshape(S, H, DH),
        v_flat.reshape(S, H, DH),
        qr_flat.reshape(S, H, DR),
        kr,
    )

    out_me = _gemm(o.reshape(S, H * DH), Wo)
    return _all_gather(out_me)


# baseline (device time: 227999 ns/iter reference)
import jax
import jax.numpy as jnp
from jax import lax
from jax.experimental import pallas as pl
from jax.experimental.pallas import tpu as pltpu

F32 = jnp.float32
BF16 = jnp.bfloat16

B, S, D = 4, 256, 4096
H, DH, DR = 32, 128, 64
DC_SH = 128
SCALE = (DH + DR) ** -0.5


def _ring_coords(rr):
    cx = jnp.where(rr >= 2, 1, 0)
    cy = jnp.where((rr == 1) | (rr == 2), 1, 0)
    return cx, cy


def _kv_body(xb_ref, xp_ref, wdkv_ref, wuk_ref, wuv_ref, k_ref, v_ref,
             send_buf, recv_buf, send_sem, recv_sem):
    my_x = lax.axis_index("x")
    my_y = lax.axis_index("y")
    wdkv = wdkv_ref[...].astype(BF16)
    wuk = wuk_ref[...].astype(BF16)
    wuv = wuv_ref[...].astype(BF16)
    cp = jnp.dot(xp_ref[...], wdkv, preferred_element_type=F32).astype(BF16)
    send_buf[0, :, :] = jnp.dot(cp, wuk, preferred_element_type=F32).astype(BF16)
    send_buf[1, :, :] = jnp.dot(cp, wuv, preferred_element_type=F32).astype(BF16)
    rdma = pltpu.make_async_remote_copy(
        src_ref=send_buf,
        dst_ref=recv_buf,
        send_sem=send_sem,
        recv_sem=recv_sem,
        device_id=(1 - my_x, my_y),
        device_id_type=pl.DeviceIdType.MESH,
    )
    rdma.start()
    cm = jnp.dot(xb_ref[...], wdkv, preferred_element_type=F32).astype(BF16)
    kp = jnp.dot(cm, wuk, preferred_element_type=F32)
    vp = jnp.dot(cm, wuv, preferred_element_type=F32)
    rdma.wait()
    k_ref[...] = (kp + recv_buf[0, :, :].astype(F32)).astype(BF16)
    v_ref[...] = (vp + recv_buf[1, :, :].astype(F32)).astype(BF16)


def _kv(xb_bf, xp_bf, Wdkv, Wuk, Wuv):
    return pl.pallas_call(
        _kv_body,
        out_shape=[
            jax.ShapeDtypeStruct((S, D), BF16),
            jax.ShapeDtypeStruct((S, D), BF16),
        ],
        in_specs=[pl.BlockSpec(memory_space=pltpu.VMEM)] * 5,
        out_specs=[pl.BlockSpec(memory_space=pltpu.VMEM)] * 2,
        scratch_shapes=[
            pltpu.VMEM((2, S, D), BF16),
            pltpu.VMEM((2, S, D), BF16),
            pltpu.SemaphoreType.DMA,
            pltpu.SemaphoreType.DMA,
        ],
    )(xb_bf, xp_bf, Wdkv, Wuk, Wuv)


def _gemm_body(x_ref, w_ref, o_ref):
    o_ref[...] = jnp.dot(
        x_ref[...], w_ref[...].astype(BF16), preferred_element_type=F32
    ).astype(BF16)


def _gemm(xb_bf, W, blk=512):
    n = W.shape[1]
    return pl.pallas_call(
        _gemm_body,
        grid=(n // blk,),
        in_specs=[
            pl.BlockSpec((S, D), lambda j: (0, 0)),
            pl.BlockSpec((D, blk), lambda j: (0, j)),
        ],
        out_specs=pl.BlockSpec((S, blk), lambda j: (0, j)),
        out_shape=jax.ShapeDtypeStruct((S, n), BF16),
    )(xb_bf, W)


def _qr_kr_body(x_ref, wqr_ref, wkr_ref, qr_ref, kr_ref):
    x = x_ref[...]
    qr_ref[...] = jnp.dot(
        x, wqr_ref[...].astype(BF16), preferred_element_type=F32
    ).astype(BF16)
    kr_ref[...] = jnp.dot(
        x, wkr_ref[...].astype(BF16), preferred_element_type=F32
    ).astype(BF16)


def _qr_kr(xb_bf, Wqr, Wkr, blk=512):
    return pl.pallas_call(
        _qr_kr_body,
        grid=(Wqr.shape[1] // blk,),
        in_specs=[
            pl.BlockSpec((S, D), lambda j: (0, 0)),
            pl.BlockSpec((D, blk), lambda j: (0, j)),
            pl.BlockSpec((D, DR), lambda j: (0, 0)),
        ],
        out_specs=[
            pl.BlockSpec((S, blk), lambda j: (0, j)),
            pl.BlockSpec((S, DR), lambda j: (0, 0)),
        ],
        out_shape=[
            jax.ShapeDtypeStruct((S, Wqr.shape[1]), BF16),
            jax.ShapeDtypeStruct((S, DR), BF16),
        ],
    )(xb_bf, Wqr, Wkr)


def _attn_body(q_ref, k_ref, v_ref, qr_ref, kr_ref, o_ref):
    qh = q_ref[...]
    kh = k_ref[...]
    vh = v_ref[...]
    qrh = qr_ref[0]
    krh = kr_ref[...]
    s = lax.dot_general(qh, kh, (((1,), (1,)), ((), ())),
                        preferred_element_type=F32)
    s = s + lax.dot_general(qrh, krh, (((1,), (1,)), ((), ())),
                            preferred_element_type=F32)
    s = s * SCALE
    m = jnp.max(s, axis=1, keepdims=True)
    e = jnp.exp(s - m)
    p = (e / jnp.sum(e, axis=1, keepdims=True)).astype(BF16)
    o_ref[...] = jnp.dot(p, vh, preferred_element_type=F32).astype(BF16)


def _attn(q, k, v, qr_hm, kr):
    return pl.pallas_call(
        _attn_body,
        grid=(H,),
        in_specs=[
            pl.BlockSpec((S, DH), lambda h: (0, h)),
            pl.BlockSpec((S, DH), lambda h: (0, h)),
            pl.BlockSpec((S, DH), lambda h: (0, h)),
            pl.BlockSpec((1, S, DR), lambda h: (h, 0, 0)),
            pl.BlockSpec((S, DR), lambda h: (0, 0)),
        ],
        out_specs=pl.BlockSpec((S, DH), lambda h: (0, h)),
        out_shape=jax.ShapeDtypeStruct((S, H * DH), BF16),
    )(q, k, v, qr_hm, kr)


def _ag_body(in_ref, out_ref, comm, send_sems, recv_sems):
    my_x = lax.axis_index("x")
    my_y = lax.axis_index("y")
    r = jnp.where(my_x == 0, my_y, 3 - my_y)
    right = _ring_coords((r + 1) % 4)
    left = _ring_coords((r + 3) % 4)

    barrier = pltpu.get_barrier_semaphore()
    for nbr in (left, right):
        pl.semaphore_signal(barrier, inc=1, device_id=nbr,
                            device_id_type=pl.DeviceIdType.MESH)
    pl.semaphore_wait(barrier, 2)

    comm[0, :, :] = in_ref[...]
    out_ref[pl.ds(r, 1), :, :] = in_ref[...].astype(F32).reshape(1, S, D)

    for h in range(3):
        rdma = pltpu.make_async_remote_copy(
            src_ref=comm.at[h],
            dst_ref=comm.at[h + 1],
            send_sem=send_sems.at[h],
            recv_sem=recv_sems.at[h],
            device_id=right,
            device_id_type=pl.DeviceIdType.MESH,
        )
        rdma.start()
        rdma.wait()
        origin = (r - h - 1) % 4
        out_ref[pl.ds(origin, 1), :, :] = (
            comm[h + 1, :, :].astype(F32).reshape(1, S, D)
        )


def _all_gather(out_me):
    return pl.pallas_call(
        _ag_body,
        out_shape=jax.ShapeDtypeStruct((B, S, D), F32),
        in_specs=[pl.BlockSpec(memory_space=pltpu.VMEM)],
        out_specs=pl.BlockSpec(memory_space=pltpu.VMEM),
        scratch_shapes=[
            pltpu.VMEM((4, S, D), BF16),
            pltpu.SemaphoreType.DMA((3,)),
            pltpu.SemaphoreType.DMA((3,)),
        ],
        compiler_params=pltpu.CompilerParams(collective_id=0),
    )(out_me)


def kernel(x, Wdkv, Wuk, Wuv, Wq, Wqr, Wkr, Wo):
    my_x = lax.axis_index("x")
    my_y = lax.axis_index("y")
    r = jnp.where(my_x == 0, my_y, 3 - my_y)
    pr = jnp.where(my_x == 1, my_y, 3 - my_y)

    xb_bf = lax.dynamic_index_in_dim(x, r, 0, keepdims=False).astype(BF16)
    xp_bf = lax.dynamic_index_in_dim(x, pr, 0, keepdims=False).astype(BF16)

    k_flat, v_flat = _kv(xb_bf, xp_bf, Wdkv, Wuk, Wuv)
    q_flat = _gemm(xb_bf, Wq)
    qr_flat, kr = _qr_kr(xb_bf, Wqr, Wkr)

    qr_hm = qr_flat.reshape(S, H, DR).transpose(1, 0, 2)
    o = _attn(q_flat, k_flat, v_flat, qr_hm, kr)

    out_me = _gemm(o, Wo)
    return _all_gather(out_me)


# device time: 170837 ns/iter; 1.3346x vs baseline; 1.3346x over previous
import jax
import jax.numpy as jnp
from jax import lax
from jax.experimental import pallas as pl
from jax.experimental.pallas import tpu as pltpu

F32 = jnp.float32
BF16 = jnp.bfloat16

B, S, D = 4, 256, 4096
H, DH, DR = 32, 128, 64
DC_SH = 128
SCALE = (DH + DR) ** -0.5


def _ring_coords(rr):
    cx = jnp.where(rr >= 2, 1, 0)
    cy = jnp.where((rr == 1) | (rr == 2), 1, 0)
    return cx, cy


def _kv_body(xb_ref, xp_ref, wdkv_ref, wuk_ref, wuv_ref, k_ref, v_ref,
             wsend, wrecv, csend, crecv, w_ssem, w_rsem, c_ssem, c_rsem):
    my_x = lax.axis_index("x")
    my_y = lax.axis_index("y")
    partner = (1 - my_x, my_y)


    wsend[0, :, :] = wuk_ref[...].astype(BF16)
    wsend[1, :, :] = wuv_ref[...].astype(BF16)
    rdma_w = pltpu.make_async_remote_copy(
        src_ref=wsend, dst_ref=wrecv, send_sem=w_ssem, recv_sem=w_rsem,
        device_id=partner, device_id_type=pl.DeviceIdType.MESH)
    rdma_w.start()

    wdkv = wdkv_ref[...].astype(BF16)
    csend[...] = jnp.dot(xp_ref[...], wdkv, preferred_element_type=F32
                         ).astype(BF16)
    rdma_c = pltpu.make_async_remote_copy(
        src_ref=csend, dst_ref=crecv, send_sem=c_ssem, recv_sem=c_rsem,
        device_id=partner, device_id_type=pl.DeviceIdType.MESH)
    rdma_c.start()

    c_me = jnp.dot(xb_ref[...], wdkv, preferred_element_type=F32).astype(BF16)
    k_loc = jnp.dot(c_me, wsend[0, :, :], preferred_element_type=F32)
    v_loc = jnp.dot(c_me, wsend[1, :, :], preferred_element_type=F32)

    rdma_c.wait()
    rdma_w.wait()
    k_ref[...] = (k_loc + jnp.dot(crecv[...], wrecv[0, :, :],
                                  preferred_element_type=F32)).astype(BF16)
    v_ref[...] = (v_loc + jnp.dot(crecv[...], wrecv[1, :, :],
                                  preferred_element_type=F32)).astype(BF16)


def _kv(xb_bf, xp_bf, Wdkv, Wuk, Wuv):
    return pl.pallas_call(
        _kv_body,
        out_shape=[
            jax.ShapeDtypeStruct((S, D), BF16),
            jax.ShapeDtypeStruct((S, D), BF16),
        ],
        in_specs=[pl.BlockSpec(memory_space=pltpu.VMEM)] * 5,
        out_specs=[pl.BlockSpec(memory_space=pltpu.VMEM)] * 2,
        scratch_shapes=[
            pltpu.VMEM((2, DC_SH, D), BF16),
            pltpu.VMEM((2, DC_SH, D), BF16),
            pltpu.VMEM((S, DC_SH), BF16),
            pltpu.VMEM((S, DC_SH), BF16),
            pltpu.SemaphoreType.DMA,
            pltpu.SemaphoreType.DMA,
            pltpu.SemaphoreType.DMA,
            pltpu.SemaphoreType.DMA,
        ],
    )(xb_bf, xp_bf, Wdkv, Wuk, Wuv)


def _gemm_body(x_ref, w_ref, o_ref):
    o_ref[...] = jnp.dot(
        x_ref[...], w_ref[...].astype(BF16), preferred_element_type=F32
    ).astype(BF16)


def _gemm(xb_bf, W, blk=512):
    n = W.shape[1]
    return pl.pallas_call(
        _gemm_body,
        grid=(n // blk,),
        in_specs=[
            pl.BlockSpec((S, D), lambda j: (0, 0)),
            pl.BlockSpec((D, blk), lambda j: (0, j)),
        ],
        out_specs=pl.BlockSpec((S, blk), lambda j: (0, j)),
        out_shape=jax.ShapeDtypeStruct((S, n), BF16),
    )(xb_bf, W)


def _qr_kr_body(x_ref, wqr_ref, wkr_ref, qr_ref, kr_ref):
    x = x_ref[...]
    qr_ref[...] = jnp.dot(
        x, wqr_ref[...].astype(BF16), preferred_element_type=F32
    ).astype(BF16)
    kr_ref[...] = jnp.dot(
        x, wkr_ref[...].astype(BF16), preferred_element_type=F32
    ).astype(BF16)


def _qr_kr(xb_bf, Wqr, Wkr, blk=512):
    return pl.pallas_call(
        _qr_kr_body,
        grid=(Wqr.shape[1] // blk,),
        in_specs=[
            pl.BlockSpec((S, D), lambda j: (0, 0)),
            pl.BlockSpec((D, blk), lambda j: (0, j)),
            pl.BlockSpec((D, DR), lambda j: (0, 0)),
        ],
        out_specs=[
            pl.BlockSpec((S, blk), lambda j: (0, j)),
            pl.BlockSpec((S, DR), lambda j: (0, 0)),
        ],
        out_shape=[
            jax.ShapeDtypeStruct((S, Wqr.shape[1]), BF16),
            jax.ShapeDtypeStruct((S, DR), BF16),
        ],
    )(xb_bf, Wqr, Wkr)


def _attn_body(q_ref, k_ref, v_ref, qr_ref, kr_ref, o_ref):
    qh = q_ref[...]
    kh = k_ref[...]
    vh = v_ref[...]
    qrh = qr_ref[0]
    krh = kr_ref[...]
    s = lax.dot_general(qh, kh, (((1,), (1,)), ((), ())),
                        preferred_element_type=F32)
    s = s + lax.dot_general(qrh, krh, (((1,), (1,)), ((), ())),
                            preferred_element_type=F32)
    s = s * SCALE
    m = jnp.max(s, axis=1, keepdims=True)
    e = jnp.exp(s - m)
    p = (e / jnp.sum(e, axis=1, keepdims=True)).astype(BF16)
    o_ref[...] = jnp.dot(p, vh, preferred_element_type=F32).astype(BF16)


def _attn(q, k, v, qr_hm, kr):
    return pl.pallas_call(
        _attn_body,
        grid=(H,),
        in_specs=[
            pl.BlockSpec((S, DH), lambda h: (0, h)),
            pl.BlockSpec((S, DH), lambda h: (0, h)),
            pl.BlockSpec((S, DH), lambda h: (0, h)),
            pl.BlockSpec((1, S, DR), lambda h: (h, 0, 0)),
            pl.BlockSpec((S, DR), lambda h: (0, 0)),
        ],
        out_specs=pl.BlockSpec((S, DH), lambda h: (0, h)),
        out_shape=jax.ShapeDtypeStruct((S, H * DH), BF16),
    )(q, k, v, qr_hm, kr)


HD = D // 2


def _ag_body(in_ref, out_ref, cw, ccw, cw_ssem, cw_rsem, ccw_ssem, ccw_rsem):
    my_x = lax.axis_index("x")
    my_y = lax.axis_index("y")
    r = jnp.where(my_x == 0, my_y, 3 - my_y)
    right = _ring_coords((r + 1) % 4)
    left = _ring_coords((r + 3) % 4)

    barrier = pltpu.get_barrier_semaphore()
    for nbr in (left, right):
        pl.semaphore_signal(barrier, inc=1, device_id=nbr,
                            device_id_type=pl.DeviceIdType.MESH)
    pl.semaphore_wait(barrier, 2)

    cw[0, :, :] = in_ref[:, :HD]
    ccw[0, :, :] = in_ref[:, HD:]
    out_ref[pl.ds(r, 1), :, :] = in_ref[...].astype(F32).reshape(1, S, D)

    for h in range(3):
        rd_cw = pltpu.make_async_remote_copy(
            src_ref=cw.at[h], dst_ref=cw.at[h + 1],
            send_sem=cw_ssem.at[h], recv_sem=cw_rsem.at[h],
            device_id=right, device_id_type=pl.DeviceIdType.MESH)
        rd_ccw = pltpu.make_async_remote_copy(
            src_ref=ccw.at[h], dst_ref=ccw.at[h + 1],
            send_sem=ccw_ssem.at[h], recv_sem=ccw_rsem.at[h],
            device_id=left, device_id_type=pl.DeviceIdType.MESH)
        rd_cw.start()
        rd_ccw.start()
        rd_cw.wait()
        rd_ccw.wait()
        o_cw = (r - h - 1) % 4
        o_ccw = (r + h + 1) % 4
        out_ref[pl.ds(o_cw, 1), :, :HD] = (
            cw[h + 1, :, :].astype(F32).reshape(1, S, HD))
        out_ref[pl.ds(o_ccw, 1), :, HD:] = (
            ccw[h + 1, :, :].astype(F32).reshape(1, S, HD))


def _all_gather(out_me):
    return pl.pallas_call(
        _ag_body,
        out_shape=jax.ShapeDtypeStruct((B, S, D), F32),
        in_specs=[pl.BlockSpec(memory_space=pltpu.VMEM)],
        out_specs=pl.BlockSpec(memory_space=pltpu.VMEM),
        scratch_shapes=[
            pltpu.VMEM((4, S, HD), BF16),
            pltpu.VMEM((4, S, HD), BF16),
            pltpu.SemaphoreType.DMA((3,)),
            pltpu.SemaphoreType.DMA((3,)),
            pltpu.SemaphoreType.DMA((3,)),
            pltpu.SemaphoreType.DMA((3,)),
        ],
        compiler_params=pltpu.CompilerParams(collective_id=0),
    )(out_me)


def kernel(x, Wdkv, Wuk, Wuv, Wq, Wqr, Wkr, Wo):
    my_x = lax.axis_index("x")
    my_y = lax.axis_index("y")
    r = jnp.where(my_x == 0, my_y, 3 - my_y)
    pr = jnp.where(my_x == 1, my_y, 3 - my_y)

    xb_bf = lax.dynamic_index_in_dim(x, r, 0, keepdims=False).astype(BF16)
    xp_bf = lax.dynamic_index_in_dim(x, pr, 0, keepdims=False).astype(BF16)

    k_flat, v_flat = _kv(xb_bf, xp_bf, Wdkv, Wuk, Wuv)
    q_flat = _gemm(xb_bf, Wq)
    qr_flat, kr = _qr_kr(xb_bf, Wqr, Wkr)

    qr_hm = qr_flat.reshape(S, H, DR).transpose(1, 0, 2)
    o = _attn(q_flat, k_flat, v_flat, qr_hm, kr)

    out_me = _gemm(o, Wo)
    return _all_gather(out_me)


# device time: 168342 ns/iter; 1.3544x vs baseline; 1.0148x over previous
import jax
import jax.numpy as jnp
from jax import lax
from jax.experimental import pallas as pl
from jax.experimental.pallas import tpu as pltpu

F32 = jnp.float32
BF16 = jnp.bfloat16

B, S, D = 4, 256, 4096
H, DH, DR = 32, 128, 64
DC_SH = 128
SCALE = (DH + DR) ** -0.5


def _ring_coords(rr):
    cx = jnp.where(rr >= 2, 1, 0)
    cy = jnp.where((rr == 1) | (rr == 2), 1, 0)
    return cx, cy


def _kv_body(xb_ref, xp_ref, wdkv_ref, wuk_ref, wuv_ref, k_ref, v_ref,
             wsend, wrecv, csend, crecv, w_ssem, w_rsem, c_ssem, c_rsem):
    my_x = lax.axis_index("x")
    my_y = lax.axis_index("y")
    partner = (1 - my_x, my_y)

    barrier = pltpu.get_barrier_semaphore()
    pl.semaphore_signal(barrier, inc=1, device_id=partner,
                        device_id_type=pl.DeviceIdType.MESH)
    pl.semaphore_wait(barrier, 1)

    wsend[0, :, :] = wuk_ref[...].astype(BF16)
    wsend[1, :, :] = wuv_ref[...].astype(BF16)
    rdma_w = pltpu.make_async_remote_copy(
        src_ref=wsend, dst_ref=wrecv, send_sem=w_ssem, recv_sem=w_rsem,
        device_id=partner, device_id_type=pl.DeviceIdType.MESH)
    rdma_w.start()

    wdkv = wdkv_ref[...].astype(BF16)
    csend[...] = jnp.dot(xp_ref[...], wdkv, preferred_element_type=F32
                         ).astype(BF16)
    rdma_c = pltpu.make_async_remote_copy(
        src_ref=csend, dst_ref=crecv, send_sem=c_ssem, recv_sem=c_rsem,
        device_id=partner, device_id_type=pl.DeviceIdType.MESH)
    rdma_c.start()

    c_me = jnp.dot(xb_ref[...], wdkv, preferred_element_type=F32).astype(BF16)
    k_loc = jnp.dot(c_me, wsend[0, :, :], preferred_element_type=F32)
    v_loc = jnp.dot(c_me, wsend[1, :, :], preferred_element_type=F32)

    rdma_c.wait()
    rdma_w.wait()
    k_ref[...] = (k_loc + jnp.dot(crecv[...], wrecv[0, :, :],
                                  preferred_element_type=F32)).astype(BF16)
    v_ref[...] = (v_loc + jnp.dot(crecv[...], wrecv[1, :, :],
                                  preferred_element_type=F32)).astype(BF16)


def _kv(xb_bf, xp_bf, Wdkv, Wuk, Wuv):
    return pl.pallas_call(
        _kv_body,
        out_shape=[
            jax.ShapeDtypeStruct((S, D), BF16),
            jax.ShapeDtypeStruct((S, D), BF16),
        ],
        in_specs=[pl.BlockSpec(memory_space=pltpu.VMEM)] * 5,
        out_specs=[pl.BlockSpec(memory_space=pltpu.VMEM)] * 2,
        scratch_shapes=[
            pltpu.VMEM((2, DC_SH, D), BF16),
            pltpu.VMEM((2, DC_SH, D), BF16),
            pltpu.VMEM((S, DC_SH), BF16),
            pltpu.VMEM((S, DC_SH), BF16),
            pltpu.SemaphoreType.DMA,
            pltpu.SemaphoreType.DMA,
            pltpu.SemaphoreType.DMA,
            pltpu.SemaphoreType.DMA,
        ],
        compiler_params=pltpu.CompilerParams(collective_id=1),
    )(xb_bf, xp_bf, Wdkv, Wuk, Wuv)


def _gemm_body(x_ref, w_ref, o_ref):
    o_ref[...] = jnp.dot(
        x_ref[...], w_ref[...].astype(BF16), preferred_element_type=F32
    ).astype(BF16)


def _gemm(xb_bf, W, blk=512):
    n = W.shape[1]
    return pl.pallas_call(
        _gemm_body,
        grid=(n // blk,),
        in_specs=[
            pl.BlockSpec((S, D), lambda j: (0, 0)),
            pl.BlockSpec((D, blk), lambda j: (0, j)),
        ],
        out_specs=pl.BlockSpec((S, blk), lambda j: (0, j)),
        out_shape=jax.ShapeDtypeStruct((S, n), BF16),
    )(xb_bf, W)


def _qr_kr_body(x_ref, wqr_ref, wkr_ref, qr_ref, kr_ref):
    x = x_ref[...]
    qr_ref[...] = jnp.dot(
        x, wqr_ref[...].astype(BF16), preferred_element_type=F32
    ).astype(BF16)
    kr_ref[...] = jnp.dot(
        x, wkr_ref[...].astype(BF16), preferred_element_type=F32
    ).astype(BF16)


def _qr_kr(xb_bf, Wqr, Wkr, blk=512):
    return pl.pallas_call(
        _qr_kr_body,
        grid=(Wqr.shape[1] // blk,),
        in_specs=[
            pl.BlockSpec((S, D), lambda j: (0, 0)),
            pl.BlockSpec((D, blk), lambda j: (0, j)),
            pl.BlockSpec((D, DR), lambda j: (0, 0)),
        ],
        out_specs=[
            pl.BlockSpec((S, blk), lambda j: (0, j)),
            pl.BlockSpec((S, DR), lambda j: (0, 0)),
        ],
        out_shape=[
            jax.ShapeDtypeStruct((S, Wqr.shape[1]), BF16),
            jax.ShapeDtypeStruct((S, DR), BF16),
        ],
    )(xb_bf, Wqr, Wkr)


def _attn_body(q_ref, k_ref, v_ref, qr_ref, kr_ref, o_ref):
    qh = q_ref[...]
    kh = k_ref[...]
    vh = v_ref[...]
    qrh = qr_ref[0]
    krh = kr_ref[...]
    s = lax.dot_general(qh, kh, (((1,), (1,)), ((), ())),
                        preferred_element_type=F32)
    s = s + lax.dot_general(qrh, krh, (((1,), (1,)), ((), ())),
                            preferred_element_type=F32)
    s = s * SCALE
    m = jnp.max(s, axis=1, keepdims=True)
    e = jnp.exp(s - m)
    p = (e / jnp.sum(e, axis=1, keepdims=True)).astype(BF16)
    o_ref[...] = jnp.dot(p, vh, preferred_element_type=F32).astype(BF16)


def _attn(q, k, v, qr_hm, kr):
    return pl.pallas_call(
        _attn_body,
        grid=(H,),
        in_specs=[
            pl.BlockSpec((S, DH), lambda h: (0, h)),
            pl.BlockSpec((S, DH), lambda h: (0, h)),
            pl.BlockSpec((S, DH), lambda h: (0, h)),
            pl.BlockSpec((1, S, DR), lambda h: (h, 0, 0)),
            pl.BlockSpec((S, DR), lambda h: (0, 0)),
        ],
        out_specs=pl.BlockSpec((S, DH), lambda h: (0, h)),
        out_shape=jax.ShapeDtypeStruct((S, H * DH), BF16),
    )(q, k, v, qr_hm, kr)


HD = D // 2


def _ag_body(in_ref, out_ref, cw, ccw, cw_ssem, cw_rsem, ccw_ssem, ccw_rsem):
    my_x = lax.axis_index("x")
    my_y = lax.axis_index("y")
    r = jnp.where(my_x == 0, my_y, 3 - my_y)
    right = _ring_coords((r + 1) % 4)
    left = _ring_coords((r + 3) % 4)

    barrier = pltpu.get_barrier_semaphore()
    for nbr in (left, right):
        pl.semaphore_signal(barrier, inc=1, device_id=nbr,
                            device_id_type=pl.DeviceIdType.MESH)
    pl.semaphore_wait(barrier, 2)

    cw[0, :, :] = in_ref[:, :HD]
    ccw[0, :, :] = in_ref[:, HD:]
    out_ref[pl.ds(r, 1), :, :] = in_ref[...].astype(F32).reshape(1, S, D)

    for h in range(3):
        rd_cw = pltpu.make_async_remote_copy(
            src_ref=cw.at[h], dst_ref=cw.at[h + 1],
            send_sem=cw_ssem.at[h], recv_sem=cw_rsem.at[h],
            device_id=right, device_id_type=pl.DeviceIdType.MESH)
        rd_ccw = pltpu.make_async_remote_copy(
            src_ref=ccw.at[h], dst_ref=ccw.at[h + 1],
            send_sem=ccw_ssem.at[h], recv_sem=ccw_rsem.at[h],
            device_id=left, device_id_type=pl.DeviceIdType.MESH)
        rd_cw.start()
        rd_ccw.start()
        rd_cw.wait()
        rd_ccw.wait()
        o_cw = (r - h - 1) % 4
        o_ccw = (r + h + 1) % 4
        out_ref[pl.ds(o_cw, 1), :, :HD] = (
            cw[h + 1, :, :].astype(F32).reshape(1, S, HD))
        out_ref[pl.ds(o_ccw, 1), :, HD:] = (
            ccw[h + 1, :, :].astype(F32).reshape(1, S, HD))


def _all_gather(out_me):
    return pl.pallas_call(
        _ag_body,
        out_shape=jax.ShapeDtypeStruct((B, S, D), F32),
        in_specs=[pl.BlockSpec(memory_space=pltpu.VMEM)],
        out_specs=pl.BlockSpec(memory_space=pltpu.VMEM),
        scratch_shapes=[
            pltpu.VMEM((4, S, HD), BF16),
            pltpu.VMEM((4, S, HD), BF16),
            pltpu.SemaphoreType.DMA((3,)),
            pltpu.SemaphoreType.DMA((3,)),
            pltpu.SemaphoreType.DMA((3,)),
            pltpu.SemaphoreType.DMA((3,)),
        ],
        compiler_params=pltpu.CompilerParams(collective_id=0),
    )(out_me)


def kernel(x, Wdkv, Wuk, Wuv, Wq, Wqr, Wkr, Wo):
    my_x = lax.axis_index("x")
    my_y = lax.axis_index("y")
    r = jnp.where(my_x == 0, my_y, 3 - my_y)
    pr = jnp.where(my_x == 1, my_y, 3 - my_y)

    xb_bf = lax.dynamic_index_in_dim(x, r, 0, keepdims=False).astype(BF16)
    xp_bf = lax.dynamic_index_in_dim(x, pr, 0, keepdims=False).astype(BF16)

    k_flat, v_flat = _kv(xb_bf, xp_bf, Wdkv, Wuk, Wuv)
    q_flat = _gemm(xb_bf, Wq)
    qr_flat, kr = _qr_kr(xb_bf, Wqr, Wkr)

    qr_hm = qr_flat.reshape(S, H, DR).transpose(1, 0, 2)
    o = _attn(q_flat, k_flat, v_flat, qr_hm, kr)

    out_me = _gemm(o, Wo)
    return _all_gather(out_me)


# device time: 157878 ns/iter; 1.4441x vs baseline; 1.0663x over previous
import jax
import jax.numpy as jnp
from jax import lax
from jax.experimental import pallas as pl
from jax.experimental.pallas import tpu as pltpu

F32 = jnp.float32
BF16 = jnp.bfloat16

B, S, D = 4, 256, 4096
H, DH, DR = 32, 128, 64
DC_SH = 128
SCALE = (DH + DR) ** -0.5


def _ring_coords(rr):
    cx = jnp.where(rr >= 2, 1, 0)
    cy = jnp.where((rr == 1) | (rr == 2), 1, 0)
    return cx, cy


def _proj_body(xb_ref, xp_ref, wdkv_ref, wuk_ref, wuv_ref, wkr_ref,
               wq_ref, wqr_ref,
               q_ref, qr_ref, kr_ref, k_ref, v_ref,
               wsend, wrecv, csend, crecv, cme,
               w_ssem, w_rsem, c_ssem, c_rsem):
    j = pl.program_id(0)
    my_x = lax.axis_index("x")
    my_y = lax.axis_index("y")
    partner = (1 - my_x, my_y)

    def _mk_w():
        return pltpu.make_async_remote_copy(
            src_ref=wsend, dst_ref=wrecv, send_sem=w_ssem, recv_sem=w_rsem,
            device_id=partner, device_id_type=pl.DeviceIdType.MESH)

    def _mk_c():
        return pltpu.make_async_remote_copy(
            src_ref=csend, dst_ref=crecv, send_sem=c_ssem, recv_sem=c_rsem,
            device_id=partner, device_id_type=pl.DeviceIdType.MESH)

    @pl.when(j == 0)
    def _():
        barrier = pltpu.get_barrier_semaphore()
        pl.semaphore_signal(barrier, inc=1, device_id=partner,
                            device_id_type=pl.DeviceIdType.MESH)
        pl.semaphore_wait(barrier, 1)
        wsend[0, :, :] = wuk_ref[...].astype(BF16)
        wsend[1, :, :] = wuv_ref[...].astype(BF16)
        _mk_w().start()
        wdkv = wdkv_ref[...].astype(BF16)
        csend[...] = jnp.dot(xp_ref[...], wdkv,
                             preferred_element_type=F32).astype(BF16)
        _mk_c().start()
        cme[...] = jnp.dot(xb_ref[...], wdkv,
                           preferred_element_type=F32).astype(BF16)

    @pl.when(j < 8)
    def _():
        q_ref[...] = jnp.dot(xb_ref[...], wq_ref[...].astype(BF16),
                             preferred_element_type=F32).astype(BF16)

    @pl.when((j >= 8) & (j < 12))
    def _():
        qr_ref[...] = jnp.dot(xb_ref[...], wqr_ref[...].astype(BF16),
                              preferred_element_type=F32).astype(BF16)

    @pl.when(j == 12)
    def _():
        kr_ref[...] = jnp.dot(xb_ref[...], wkr_ref[...].astype(BF16),
                              preferred_element_type=F32).astype(BF16)
        c_me = cme[...]
        k_loc = jnp.dot(c_me, wsend[0, :, :], preferred_element_type=F32)
        v_loc = jnp.dot(c_me, wsend[1, :, :], preferred_element_type=F32)
        _mk_c().wait()
        _mk_w().wait()
        k_ref[...] = (k_loc + jnp.dot(crecv[...], wrecv[0, :, :],
                                      preferred_element_type=F32)).astype(BF16)
        v_ref[...] = (v_loc + jnp.dot(crecv[...], wrecv[1, :, :],
                                      preferred_element_type=F32)).astype(BF16)


def _proj(xb_bf, xp_bf, Wdkv, Wuk, Wuv, Wkr, Wq, Wqr, blk=512):
    c7 = lambda j: (0, jnp.clip(j, 0, 7))
    c3 = lambda j: (0, jnp.clip(j - 8, 0, 3))
    return pl.pallas_call(
        _proj_body,
        grid=(13,),
        in_specs=[
            pl.BlockSpec((S, D), lambda j: (0, 0)),
            pl.BlockSpec((S, D), lambda j: (0, 0)),
            pl.BlockSpec((D, DC_SH), lambda j: (0, 0)),
            pl.BlockSpec((DC_SH, D), lambda j: (0, 0)),
            pl.BlockSpec((DC_SH, D), lambda j: (0, 0)),
            pl.BlockSpec((D, DR), lambda j: (0, 0)),
            pl.BlockSpec((D, blk), c7),
            pl.BlockSpec((D, blk), c3),
        ],
        out_specs=[
            pl.BlockSpec((S, blk), c7),
            pl.BlockSpec((S, blk), c3),
            pl.BlockSpec((S, DR), lambda j: (0, 0)),
            pl.BlockSpec((S, D), lambda j: (0, 0)),
            pl.BlockSpec((S, D), lambda j: (0, 0)),
        ],
        out_shape=[
            jax.ShapeDtypeStruct((S, D), BF16),
            jax.ShapeDtypeStruct((S, 4 * blk), BF16),
            jax.ShapeDtypeStruct((S, DR), BF16),
            jax.ShapeDtypeStruct((S, D), BF16),
            jax.ShapeDtypeStruct((S, D), BF16),
        ],
        scratch_shapes=[
            pltpu.VMEM((2, DC_SH, D), BF16),
            pltpu.VMEM((2, DC_SH, D), BF16),
            pltpu.VMEM((S, DC_SH), BF16),
            pltpu.VMEM((S, DC_SH), BF16),
            pltpu.VMEM((S, DC_SH), BF16),
            pltpu.SemaphoreType.DMA,
            pltpu.SemaphoreType.DMA,
            pltpu.SemaphoreType.DMA,
            pltpu.SemaphoreType.DMA,
        ],
        compiler_params=pltpu.CompilerParams(
            collective_id=1, vmem_limit_bytes=100 * 1024 * 1024),
    )(xb_bf, xp_bf, Wdkv, Wuk, Wuv, Wkr, Wq, Wqr)


def _gemm_body(x_ref, w_ref, o_ref):
    o_ref[...] = jnp.dot(
        x_ref[...], w_ref[...].astype(BF16), preferred_element_type=F32
    ).astype(BF16)


def _gemm(xb_bf, W, blk=512):
    n = W.shape[1]
    return pl.pallas_call(
        _gemm_body,
        grid=(n // blk,),
        in_specs=[
            pl.BlockSpec((S, D), lambda j: (0, 0)),
            pl.BlockSpec((D, blk), lambda j: (0, j)),
        ],
        out_specs=pl.BlockSpec((S, blk), lambda j: (0, j)),
        out_shape=jax.ShapeDtypeStruct((S, n), BF16),
    )(xb_bf, W)


def _qr_kr_body(x_ref, wqr_ref, wkr_ref, qr_ref, kr_ref):
    x = x_ref[...]
    qr_ref[...] = jnp.dot(
        x, wqr_ref[...].astype(BF16), preferred_element_type=F32
    ).astype(BF16)
    kr_ref[...] = jnp.dot(
        x, wkr_ref[...].astype(BF16), preferred_element_type=F32
    ).astype(BF16)


def _qr_kr(xb_bf, Wqr, Wkr, blk=512):
    return pl.pallas_call(
        _qr_kr_body,
        grid=(Wqr.shape[1] // blk,),
        in_specs=[
            pl.BlockSpec((S, D), lambda j: (0, 0)),
            pl.BlockSpec((D, blk), lambda j: (0, j)),
            pl.BlockSpec((D, DR), lambda j: (0, 0)),
        ],
        out_specs=[
            pl.BlockSpec((S, blk), lambda j: (0, j)),
            pl.BlockSpec((S, DR), lambda j: (0, 0)),
        ],
        out_shape=[
            jax.ShapeDtypeStruct((S, Wqr.shape[1]), BF16),
            jax.ShapeDtypeStruct((S, DR), BF16),
        ],
    )(xb_bf, Wqr, Wkr)


def _attn_body(q_ref, k_ref, v_ref, qr_ref, kr_ref, o_ref):
    qh = q_ref[...]
    kh = k_ref[...]
    vh = v_ref[...]
    qrh = qr_ref[0]
    krh = kr_ref[...]
    s = lax.dot_general(qh, kh, (((1,), (1,)), ((), ())),
                        preferred_element_type=F32)
    s = s + lax.dot_general(qrh, krh, (((1,), (1,)), ((), ())),
                            preferred_element_type=F32)
    s = s * SCALE
    m = jnp.max(s, axis=1, keepdims=True)
    e = jnp.exp(s - m)
    p = (e / jnp.sum(e, axis=1, keepdims=True)).astype(BF16)
    o_ref[...] = jnp.dot(p, vh, preferred_element_type=F32).astype(BF16)


def _attn(q, k, v, qr_hm, kr):
    return pl.pallas_call(
        _attn_body,
        grid=(H,),
        in_specs=[
            pl.BlockSpec((S, DH), lambda h: (0, h)),
            pl.BlockSpec((S, DH), lambda h: (0, h)),
            pl.BlockSpec((S, DH), lambda h: (0, h)),
            pl.BlockSpec((1, S, DR), lambda h: (h, 0, 0)),
            pl.BlockSpec((S, DR), lambda h: (0, 0)),
        ],
        out_specs=pl.BlockSpec((S, DH), lambda h: (0, h)),
        out_shape=jax.ShapeDtypeStruct((S, H * DH), BF16),
    )(q, k, v, qr_hm, kr)


HD = D // 2


def _wo_ag_body(o_ref, wo_ref, out_ref, me, cw, ccw,
                cw_ssem, cw_rsem, ccw_ssem, ccw_rsem):
    j = pl.program_id(0)
    my_x = lax.axis_index("x")
    my_y = lax.axis_index("y")
    r = jnp.where(my_x == 0, my_y, 3 - my_y)
    right = _ring_coords((r + 1) % 4)
    left = _ring_coords((r + 3) % 4)

    def _mk(h, comm, ssem, rsem, dev):
        return pltpu.make_async_remote_copy(
            src_ref=comm.at[h], dst_ref=comm.at[h + 1],
            send_sem=ssem.at[h], recv_sem=rsem.at[h],
            device_id=dev, device_id_type=pl.DeviceIdType.MESH)

    @pl.when(j == 0)
    def _():
        barrier = pltpu.get_barrier_semaphore()
        for nbr in (left, right):
            pl.semaphore_signal(barrier, inc=1, device_id=nbr,
                                device_id_type=pl.DeviceIdType.MESH)
        pl.semaphore_wait(barrier, 2)

    blk = 512
    me[:, pl.ds(j * blk, blk)] = jnp.dot(
        o_ref[...], wo_ref[...].astype(BF16),
        preferred_element_type=F32).astype(BF16)

    @pl.when(j == 3)
    def _():
        cw[0, :, :] = me[:, :HD]
        _mk(0, cw, cw_ssem, cw_rsem, right).start()

    @pl.when(j == 7)
    def _():
        ccw[0, :, :] = me[:, HD:]
        _mk(0, ccw, ccw_ssem, ccw_rsem, left).start()
        out_ref[pl.ds(r, 1), :, :] = me[...].astype(F32).reshape(1, S, D)
        for h in range(3):
            _mk(h, cw, cw_ssem, cw_rsem, right).wait()
            if h < 2:
                _mk(h + 1, cw, cw_ssem, cw_rsem, right).start()
            o_cw = (r - h - 1) % 4
            out_ref[pl.ds(o_cw, 1), :, :HD] = (
                cw[h + 1, :, :].astype(F32).reshape(1, S, HD))
            _mk(h, ccw, ccw_ssem, ccw_rsem, left).wait()
            if h < 2:
                _mk(h + 1, ccw, ccw_ssem, ccw_rsem, left).start()
            o_ccw = (r + h + 1) % 4
            out_ref[pl.ds(o_ccw, 1), :, HD:] = (
                ccw[h + 1, :, :].astype(F32).reshape(1, S, HD))


def _wo_ag(o, Wo, blk=512):
    return pl.pallas_call(
        _wo_ag_body,
        grid=(8,),
        in_specs=[
            pl.BlockSpec((S, D), lambda j: (0, 0)),
            pl.BlockSpec((D, blk), lambda j: (0, j)),
        ],
        out_specs=pl.BlockSpec((B, S, D), lambda j: (0, 0, 0)),
        out_shape=jax.ShapeDtypeStruct((B, S, D), F32),
        scratch_shapes=[
            pltpu.VMEM((S, D), BF16),
            pltpu.VMEM((4, S, HD), BF16),
            pltpu.VMEM((4, S, HD), BF16),
            pltpu.SemaphoreType.DMA((3,)),
            pltpu.SemaphoreType.DMA((3,)),
            pltpu.SemaphoreType.DMA((3,)),
            pltpu.SemaphoreType.DMA((3,)),
        ],
        compiler_params=pltpu.CompilerParams(collective_id=0),
    )(o, Wo)


def kernel(x, Wdkv, Wuk, Wuv, Wq, Wqr, Wkr, Wo):
    my_x = lax.axis_index("x")
    my_y = lax.axis_index("y")
    r = jnp.where(my_x == 0, my_y, 3 - my_y)
    pr = jnp.where(my_x == 1, my_y, 3 - my_y)

    xb_bf = lax.dynamic_index_in_dim(x, r, 0, keepdims=False).astype(BF16)
    xp_bf = lax.dynamic_index_in_dim(x, pr, 0, keepdims=False).astype(BF16)

    q_flat, qr_flat, kr, k_flat, v_flat = _proj(
        xb_bf, xp_bf, Wdkv, Wuk, Wuv, Wkr, Wq, Wqr)

    qr_hm = qr_flat.reshape(S, H, DR).transpose(1, 0, 2)
    o = _attn(q_flat, k_flat, v_flat, qr_hm, kr)

    return _wo_ag(o, Wo)
